# baseline (speedup 1.0000x reference)
import functools

import jax
import jax.numpy as jnp
from jax.experimental import pallas as pl
from jax.experimental.pallas import tpu as pltpu

BM = 256
FT = 512


def _glu_block_kernel(nf, be_ref, xs_ref, ss_ref, wg_ref, wu_ref, wd_ref, o_ref):
    f = pl.program_id(1)
    xb = xs_ref[...].astype(jnp.bfloat16)
    g = jnp.dot(xb, wg_ref[0].astype(jnp.bfloat16), preferred_element_type=jnp.float32)
    u = jnp.dot(xb, wu_ref[0].astype(jnp.bfloat16), preferred_element_type=jnp.float32)
    h = ((g * jax.nn.sigmoid(g)) * u).astype(jnp.bfloat16)
    acc = jnp.dot(h, wd_ref[0].astype(jnp.bfloat16), preferred_element_type=jnp.float32)

    @pl.when(f == 0)
    def _():
        o_ref[...] = acc

    @pl.when(f > 0)
    def _():
        o_ref[...] = o_ref[...] + acc

    @pl.when(f == nf - 1)
    def _():
        o_ref[...] = o_ref[...] * ss_ref[...]


def _grouped_glu(xs, ss_col, Wg, Wu, Wd, block_expert, nb, nf):
    P, D = xs.shape
    F = Wg.shape[2]
    grid_spec = pltpu.PrefetchScalarGridSpec(
        num_scalar_prefetch=1,
        grid=(nb, nf),
        in_specs=[
            pl.BlockSpec((BM, D), lambda b, f, be: (b, 0)),
            pl.BlockSpec((BM, 1), lambda b, f, be: (b, 0)),
            pl.BlockSpec((1, D, FT), lambda b, f, be: (be[b], 0, f)),
            pl.BlockSpec((1, D, FT), lambda b, f, be: (be[b], 0, f)),
            pl.BlockSpec((1, FT, D), lambda b, f, be: (be[b], f, 0)),
        ],
        out_specs=pl.BlockSpec((BM, D), lambda b, f, be: (b, 0)),
    )
    return pl.pallas_call(
        functools.partial(_glu_block_kernel, nf),
        grid_spec=grid_spec,
        out_shape=jax.ShapeDtypeStruct((P, D), jnp.float32),
        compiler_params=pltpu.CompilerParams(
            dimension_semantics=("arbitrary", "arbitrary"),
        ),
    )(block_expert, xs, ss_col, Wg, Wu, Wd)


def kernel(x, topK_indices, topK_scores, Wg, Wu, Wd):
    T, D = x.shape
    _, K = topK_indices.shape
    E, _, F = Wg.shape
    S = T * K
    P = S + E * BM
    NB = P // BM
    NF = F // FT

    idx = topK_indices.reshape(-1).astype(jnp.int32)
    scores = topK_scores.reshape(-1)

    counts = jnp.bincount(idx, length=E)
    sizes = ((counts + BM - 1) // BM) * BM
    ends = jnp.cumsum(sizes)
    starts = ends - sizes
    seg_begin = jnp.cumsum(counts) - counts

    order = jnp.argsort(idx, stable=True)
    sorted_e = idx[order]
    pos_sorted = (starts[sorted_e] + (jnp.arange(S) - seg_begin[sorted_e])).astype(jnp.int32)
    slot_pos = jnp.zeros((S,), jnp.int32).at[order].set(pos_sorted)
    gidx = jnp.zeros((P,), jnp.int32).at[pos_sorted].set((order // K).astype(jnp.int32))
    ss = jnp.zeros((P,), jnp.float32).at[pos_sorted].set(scores[order])
    block_expert = jnp.minimum(
        jnp.searchsorted(ends, jnp.arange(NB, dtype=jnp.int32) * BM, side="right"),
        E - 1,
    ).astype(jnp.int32)

    xs = x[gidx]
    out_rows = _grouped_glu(xs, ss[:, None], Wg, Wu, Wd, block_expert, NB, NF)
    y = out_rows[slot_pos].reshape(T, K, D).sum(axis=1)
    return y


# speedup vs baseline: 1.3168x; 1.3168x over previous
import functools

import jax
import jax.numpy as jnp
from jax.experimental import pallas as pl
from jax.experimental.pallas import tpu as pltpu

BM = 256
FT = 2048


def _glu_block_kernel(nf, be_ref, xs_ref, ss_ref, wg_ref, wu_ref, wd_ref, o_ref):
    f = pl.program_id(1)
    xb = xs_ref[...].astype(jnp.bfloat16)
    g = jnp.dot(xb, wg_ref[0].astype(jnp.bfloat16), preferred_element_type=jnp.float32)
    u = jnp.dot(xb, wu_ref[0].astype(jnp.bfloat16), preferred_element_type=jnp.float32)
    h = ((g * jax.nn.sigmoid(g)) * u).astype(jnp.bfloat16)
    acc = jnp.dot(h, wd_ref[0].astype(jnp.bfloat16), preferred_element_type=jnp.float32)

    @pl.when(f == 0)
    def _():
        o_ref[...] = acc

    @pl.when(f > 0)
    def _():
        o_ref[...] = o_ref[...] + acc

    @pl.when(f == nf - 1)
    def _():
        o_ref[...] = o_ref[...] * ss_ref[...]


def _grouped_glu(xs, ss_col, Wg, Wu, Wd, block_expert, nb, nf):
    P, D = xs.shape
    F = Wg.shape[2]
    grid_spec = pltpu.PrefetchScalarGridSpec(
        num_scalar_prefetch=1,
        grid=(nb, nf),
        in_specs=[
            pl.BlockSpec((BM, D), lambda b, f, be: (b, 0)),
            pl.BlockSpec((BM, 1), lambda b, f, be: (b, 0)),
            pl.BlockSpec((1, D, FT), lambda b, f, be: (be[b], 0, f)),
            pl.BlockSpec((1, D, FT), lambda b, f, be: (be[b], 0, f)),
            pl.BlockSpec((1, FT, D), lambda b, f, be: (be[b], f, 0)),
        ],
        out_specs=pl.BlockSpec((BM, D), lambda b, f, be: (b, 0)),
    )
    return pl.pallas_call(
        functools.partial(_glu_block_kernel, nf),
        grid_spec=grid_spec,
        out_shape=jax.ShapeDtypeStruct((P, D), jnp.float32),
        compiler_params=pltpu.CompilerParams(
            dimension_semantics=("arbitrary", "arbitrary"),
        ),
    )(block_expert, xs, ss_col, Wg, Wu, Wd)


def kernel(x, topK_indices, topK_scores, Wg, Wu, Wd):
    T, D = x.shape
    _, K = topK_indices.shape
    E, _, F = Wg.shape
    S = T * K
    P = S + E * BM
    NB = P // BM
    NF = F // FT

    idx = topK_indices.reshape(-1).astype(jnp.int32)
    scores = topK_scores.reshape(-1)

    counts = jnp.bincount(idx, length=E)
    sizes = ((counts + BM - 1) // BM) * BM
    ends = jnp.cumsum(sizes)
    starts = ends - sizes
    seg_begin = jnp.cumsum(counts) - counts

    order = jnp.argsort(idx, stable=True)
    sorted_e = idx[order]
    pos_sorted = (starts[sorted_e] + (jnp.arange(S) - seg_begin[sorted_e])).astype(jnp.int32)
    slot_pos = jnp.zeros((S,), jnp.int32).at[order].set(pos_sorted)
    gidx = jnp.zeros((P,), jnp.int32).at[pos_sorted].set((order // K).astype(jnp.int32))
    ss = jnp.zeros((P,), jnp.float32).at[pos_sorted].set(scores[order])
    block_expert = jnp.minimum(
        jnp.searchsorted(ends, jnp.arange(NB, dtype=jnp.int32) * BM, side="right"),
        E - 1,
    ).astype(jnp.int32)

    xs = x[gidx]
    out_rows = _grouped_glu(xs, ss[:, None], Wg, Wu, Wd, block_expert, NB, NF)
    y = out_rows[slot_pos].reshape(T, K, D).sum(axis=1)
    return y
